# baseline (speedup 1.0000x reference)
import functools

import jax
import jax.numpy as jnp
from jax import lax
from jax.experimental import pallas as pl
from jax.experimental.pallas import tpu as pltpu
from jax.experimental.pallas import tpu_sc as plsc

_NUM_CORES = 2
_NUM_SUBCORES = 16
_NUM_WORKERS = _NUM_CORES * _NUM_SUBCORES
_LANES = 16


def _emb_body(n, d, seq_per_w, n_half,
              x_hbm, tok_hbm, pos_hbm, out_hbm, idx_v, rows_v, pos_v, sem):
  c = lax.axis_index("c")
  s = lax.axis_index("s")
  wid = s * _NUM_CORES + c

  pltpu.sync_copy(pos_hbm, pos_v)

  @pl.loop(0, seq_per_w)
  def _seq(i):
    row = wid * seq_per_w + i
    pltpu.sync_copy(x_hbm.at[row], idx_v)
    cp0 = pltpu.async_copy(tok_hbm.at[idx_v.at[0]], rows_v.at[pl.ds(0, n_half)], sem)
    cp1 = pltpu.async_copy(tok_hbm.at[idx_v.at[1]], rows_v.at[pl.ds(n_half, n_half)], sem)
    cp0.wait()
    cp1.wait()

    @pl.loop(0, n)
    def _row(j):
      for k in range(d // _LANES):
        sl = pl.ds(k * _LANES, _LANES)
        plsc.addupdate(rows_v.at[j, sl], pos_v[j, sl])

    pltpu.sync_copy(rows_v, out_hbm.at[pl.ds(row * n, n)])


def kernel(X, token_table, pos_table):
  b, n = X.shape
  v, d = token_table.shape
  assert b % _NUM_WORKERS == 0 and d % _LANES == 0
  seq_per_w = b // _NUM_WORKERS
  n_half = n // 2
  assert n_half * 2 == n and n_half <= 128

  x3 = X.astype(jnp.int32).reshape(b, 2, n_half)
  mesh = plsc.VectorSubcoreMesh(core_axis_name="c", subcore_axis_name="s")

  emb = pl.kernel(
      functools.partial(_emb_body, n, d, seq_per_w, n_half),
      out_type=jax.ShapeDtypeStruct((b * n, d), jnp.float32),
      mesh=mesh,
      scratch_types=[
          pltpu.VMEM((2, n_half), jnp.int32),
          pltpu.VMEM((n, d), jnp.float32),
          pltpu.VMEM((n, d), jnp.float32),
          pltpu.SemaphoreType.DMA,
      ],
      compiler_params=pltpu.CompilerParams(use_tc_tiling_on_sc=False),
  )
  out = emb(x3, token_table, pos_table)
  return out.reshape(b, n, d)


# speedup vs baseline: 1.1909x; 1.1909x over previous
import functools

import jax
import jax.numpy as jnp
from jax import lax
from jax.experimental import pallas as pl
from jax.experimental.pallas import tpu as pltpu
from jax.experimental.pallas import tpu_sc as plsc

_NUM_CORES = 2
_NUM_SUBCORES = 16
_NUM_WORKERS = _NUM_CORES * _NUM_SUBCORES
_LANES = 16
_NBUF = 2


def _emb_body(n, d, seq_per_w, n_half,
              x_hbm, tok_hbm, pos_hbm, out_hbm,
              idx_v, rows_v, obuf_v, pos_v, gsem0, gsem1, ssem0, ssem1):
  c = lax.axis_index("c")
  s = lax.axis_index("s")
  wid = s * _NUM_CORES + c
  base_seq = wid * seq_per_w
  gsems = (gsem0, gsem1)
  ssems = (ssem0, ssem1)
  n_outer = seq_per_w // _NBUF

  pltpu.sync_copy(pos_hbm, pos_v)
  pltpu.sync_copy(x_hbm.at[pl.ds(base_seq, seq_per_w)], idx_v)

  def issue_gather(i_local, b):
    rows_b = rows_v.at[b]
    pltpu.async_copy(tok_hbm.at[idx_v.at[i_local, 0]],
                     rows_b.at[pl.ds(0, n_half)], gsems[b])
    pltpu.async_copy(tok_hbm.at[idx_v.at[i_local, 1]],
                     rows_b.at[pl.ds(n_half, n_half)], gsems[b])

  def drain_gather(b):
    pltpu.make_async_copy(tok_hbm.at[pl.ds(0, n)], rows_v.at[b],
                          gsems[b]).wait()

  def drain_scatter(b):
    pltpu.make_async_copy(obuf_v.at[b], out_hbm.at[pl.ds(0, n)],
                          ssems[b]).wait()

  for b in range(_NBUF):
    issue_gather(jnp.int32(b), b)

  @pl.loop(0, n_outer)
  def _outer(o):
    for b in range(_NBUF):
      i_local = o * _NBUF + b
      @pl.when(o >= 1)
      def _():
        drain_scatter(b)
      drain_gather(b)

      @plsc.parallel_loop(0, n, unroll=4)
      def _row(j):
        for k in range(d // _LANES):
          sl = pl.ds(k * _LANES, _LANES)
          obuf_v.at[b][j, sl] = rows_v.at[b][j, sl] + pos_v[j, sl]

      @pl.when(o < n_outer - 1)
      def _():
        issue_gather(i_local + _NBUF, b)
      pltpu.async_copy(obuf_v.at[b],
                       out_hbm.at[pl.ds((base_seq + i_local) * n, n)],
                       ssems[b])

  for b in range(_NBUF):
    drain_scatter(b)


def kernel(X, token_table, pos_table):
  b, n = X.shape
  v, d = token_table.shape
  assert b % (_NUM_WORKERS * _NBUF) == 0 and d % _LANES == 0
  seq_per_w = b // _NUM_WORKERS
  n_half = n // 2
  assert n_half * 2 == n and n_half <= 128

  x3 = X.astype(jnp.int32).reshape(b, 2, n_half)
  mesh = plsc.VectorSubcoreMesh(core_axis_name="c", subcore_axis_name="s")

  emb = pl.kernel(
      functools.partial(_emb_body, n, d, seq_per_w, n_half),
      out_type=jax.ShapeDtypeStruct((b * n, d), jnp.float32),
      mesh=mesh,
      scratch_types=[
          pltpu.VMEM((seq_per_w, 2, n_half), jnp.int32),
          pltpu.VMEM((_NBUF, n, d), jnp.float32),
          pltpu.VMEM((_NBUF, n, d), jnp.float32),
          pltpu.VMEM((n, d), jnp.float32),
          pltpu.SemaphoreType.DMA,
          pltpu.SemaphoreType.DMA,
          pltpu.SemaphoreType.DMA,
          pltpu.SemaphoreType.DMA,
      ],
      compiler_params=pltpu.CompilerParams(use_tc_tiling_on_sc=False),
  )
  out = emb(x3, token_table, pos_table)
  return out.reshape(b, n, d)
